# Initial kernel scaffold; baseline (speedup 1.0000x reference)
import functools

import jax
import jax.numpy as jnp
from jax import lax
from jax.experimental import pallas as pl
from jax.experimental.pallas import tpu as pltpu
from jax.experimental.pallas import tpu_sc as plsc

E = 64
D = 1024
DFF = 1024
T = 4096
BLK = 128
NCHUNK = 96
NP = NCHUNK * BLK
NTILE = 16
TPT = T // NTILE
NGROUP = TPT // 16
NC = 2
NW = NC * NTILE
GPW = NP // NW
GCH = 96
CPW = T // NW
CCH = 64

_mesh = plsc.VectorSubcoreMesh(core_axis_name="c", subcore_axis_name="s")


def _i32(x):
    return jnp.asarray(x, jnp.int32)


def _lane():
    return lax.iota(jnp.int32, 16)


@functools.partial(
    pl.kernel,
    mesh=_mesh,
    out_type=[
        jax.ShapeDtypeStruct((T,), jnp.int32),
        jax.ShapeDtypeStruct((NP,), jnp.int32),
        jax.ShapeDtypeStruct((NP,), jnp.float32),
        jax.ShapeDtypeStruct((NCHUNK,), jnp.int32),
    ],
    scratch_types=[
        pltpu.VMEM((TPT, E), jnp.float32),
        pltpu.VMEM((E,), jnp.float32),
        pltpu.VMEM((TPT,), jnp.int32),
        pltpu.VMEM((TPT,), jnp.int32),
        pltpu.VMEM((TPT,), jnp.float32),
        pltpu.VMEM((TPT,), jnp.int32),
        pltpu.VMEM((E,), jnp.int32),
        pltpu.VMEM((NTILE, E), jnp.int32),
        pltpu.VMEM((E,), jnp.int32),
        pltpu.VMEM((E,), jnp.int32),
        pltpu.VMEM((NP // NTILE,), jnp.int32),
        pltpu.VMEM((NP // NTILE,), jnp.float32),
        pltpu.VMEM((128,), jnp.int32),
        pltpu.VMEM((128,), jnp.int32),
        pltpu.VMEM((128,), jnp.float32),
        pltpu.VMEM((NCHUNK,), jnp.int32),
        pltpu.VMEM_SHARED((NTILE, E), jnp.int32),
        pltpu.VMEM_SHARED((NP,), jnp.int32),
        pltpu.VMEM_SHARED((NP,), jnp.float32),
    ],
)
def _routing_kernel(logits_hbm, scale_hbm, rank_hbm, sids_hbm, wdisp_hbm,
                    cmap_hbm, lbuf, scalebuf, top1buf, rlocbuf, wtbuf,
                    rankbuf, cntbuf, allcnt, mybase, endbuf, zbuf_i, zbuf_f,
                    idxbuf, idsbuf, wsrcbuf, cmapbuf, sh_cnt, sh_sids,
                    sh_wdisp):
    c = lax.axis_index("c")
    s = lax.axis_index("s")

    @pl.when(c == 0)
    def _():
        lane = _lane()
        zpt = NP // NTILE

        pltpu.sync_copy(logits_hbm.at[pl.ds(s * TPT, TPT)], lbuf)
        pltpu.sync_copy(scale_hbm, scalebuf)

        for k in range(E // 16):
            cntbuf[pl.ds(k * 16, 16)] = jnp.zeros((16,), jnp.int32)
        for k in range(zpt // 16):
            zbuf_i[pl.ds(k * 16, 16)] = jnp.zeros((16,), jnp.int32)
            zbuf_f[pl.ds(k * 16, 16)] = jnp.zeros((16,), jnp.float32)
        pltpu.sync_copy(zbuf_i, sh_sids.at[pl.ds(s * zpt, zpt)])
        pltpu.sync_copy(zbuf_f, sh_wdisp.at[pl.ds(s * zpt, zpt)])

        def group_body(g, _):
            rows = g * 16 + lane
            m0 = jnp.full((16,), -jnp.inf, jnp.float32)
            be0 = jnp.zeros((16,), jnp.int32)

            def exp_body(e, carry):
                m, be = carry
                v = plsc.load_gather(lbuf, [rows, jnp.broadcast_to(e, (16,))])
                upd = v > m
                return jnp.where(upd, v, m), jnp.where(upd, e, be)

            m, be = lax.fori_loop(0, E, exp_body, (m0, be0))
            top1buf[pl.ds(g * 16, 16)] = be
            wtbuf[pl.ds(g * 16, 16)] = plsc.load_gather(scalebuf, [be])

            wrk = jnp.zeros((16,), jnp.int32)
            aft = jnp.zeros((16,), jnp.int32)
            for j in range(16):
                bj = plsc.load_gather(
                    top1buf, [jnp.broadcast_to(g * 16 + j, (16,))])
                eqv = be == bj
                wrk = wrk + jnp.where(eqv & (lane > j), 1, 0)
                aft = aft + jnp.where(eqv & (lane < j), 1, 0)
            base = plsc.load_gather(cntbuf, [be])
            rloc = base + wrk
            plsc.store_scatter(cntbuf, [be], rloc + 1, mask=aft == 0)
            rlocbuf[pl.ds(g * 16, 16)] = rloc
            return 0

        lax.fori_loop(0, NGROUP, group_body, 0)

        pltpu.sync_copy(cntbuf, sh_cnt.at[s])
        plsc.subcore_barrier()
        pltpu.sync_copy(sh_cnt, allcnt)

        carry = jnp.zeros((), jnp.int32)
        for k in range(E // 16):
            sl = pl.ds(k * 16, 16)
            tot = jnp.zeros((16,), jnp.int32)
            tpre = jnp.zeros((16,), jnp.int32)
            for t in range(NTILE):
                v = allcnt[t, sl]
                tot = tot + v
                tpre = tpre + jnp.where(_i32(t) < s, v, 0)
            padded = ((tot + (BLK - 1)) >> 7) << 7
            cum = jnp.cumsum(padded)
            excl = cum - padded + carry
            carry = carry + jnp.sum(padded)
            mybase[sl] = excl + tpre
            endbuf[sl] = excl + padded

        def rank_body(g, _):
            sl = pl.ds(g * 16, 16)
            be = top1buf[sl]
            rankbuf[sl] = plsc.load_gather(mybase, [be]) + rlocbuf[sl]
            return 0

        lax.fori_loop(0, NGROUP, rank_body, 0)
        pltpu.sync_copy(rankbuf, rank_hbm.at[pl.ds(s * TPT, TPT)])

        plsc.subcore_barrier()
        for h in range(TPT // 128):
            for q in range(8):
                src = pl.ds(h * 128 + q * 16, 16)
                dst = pl.ds(q * 16, 16)
                idxbuf[dst] = rankbuf[src]
                idsbuf[dst] = s * TPT + h * 128 + q * 16 + lane
                wsrcbuf[dst] = wtbuf[src]
            pltpu.sync_copy(idsbuf, sh_sids.at[idxbuf])
            pltpu.sync_copy(wsrcbuf, sh_wdisp.at[idxbuf])
        plsc.subcore_barrier()
        pltpu.sync_copy(sh_sids.at[pl.ds(s * zpt, zpt)], zbuf_i)
        pltpu.sync_copy(zbuf_i, sids_hbm.at[pl.ds(s * zpt, zpt)])
        pltpu.sync_copy(sh_wdisp.at[pl.ds(s * zpt, zpt)], zbuf_f)
        pltpu.sync_copy(zbuf_f, wdisp_hbm.at[pl.ds(s * zpt, zpt)])

        @pl.when(s == 0)
        def _():
            for cv in range(NCHUNK // 16):
                rowvec = (cv * 16 + lane) * BLK
                acc = jnp.zeros((16,), jnp.int32)
                for e in range(E):
                    end_e = plsc.load_gather(
                        endbuf, [jnp.broadcast_to(e, (16,))])
                    acc = acc + jnp.where(end_e <= rowvec, 1, 0)
                cmapbuf[pl.ds(cv * 16, 16)] = jnp.minimum(acc, E - 1)
            pltpu.sync_copy(cmapbuf, cmap_hbm)


@functools.partial(
    pl.kernel,
    mesh=_mesh,
    out_type=jax.ShapeDtypeStruct((NP, D), jnp.float32),
    scratch_types=[
        pltpu.VMEM((GCH,), jnp.int32),
        pltpu.VMEM((GCH, D), jnp.float32),
        pltpu.SemaphoreType.DMA,
    ],
)
def _dispatch_kernel(hid_hbm, sids_hbm, disp_hbm, idxv, rowsv, sem):
    wid = lax.axis_index("s") * NC + lax.axis_index("c")
    for p in range(GPW // GCH):
        base = wid * GPW + p * GCH
        pltpu.sync_copy(sids_hbm.at[pl.ds(base, GCH)], idxv)
        pltpu.async_copy(hid_hbm.at[idxv], rowsv, sem).wait()
        pltpu.sync_copy(rowsv, disp_hbm.at[pl.ds(base, GCH)])


def _mlp_body(cmap_ref, x_ref, w1_ref, w3_ref, w2_ref, wd_ref, o_ref):
    x = x_ref[...]
    a = lax.dot_general(x, w1_ref[0], (((1,), (1,)), ((), ())),
                        preferred_element_type=jnp.float32)
    b = lax.dot_general(x, w3_ref[0], (((1,), (1,)), ((), ())),
                        preferred_element_type=jnp.float32)
    h = jax.nn.gelu(a) * b
    y = lax.dot_general(h, w2_ref[0], (((1,), (1,)), ((), ())),
                        preferred_element_type=jnp.float32)
    o_ref[...] = y * wd_ref[0, 0][:, None]


def _mlp_call(cmap, disp, w1, w3, w2, wd3):
    grid_spec = pltpu.PrefetchScalarGridSpec(
        num_scalar_prefetch=1,
        grid=(NCHUNK,),
        in_specs=[
            pl.BlockSpec((BLK, D), lambda i, cm: (i, 0)),
            pl.BlockSpec((1, DFF, D), lambda i, cm: (cm[i], 0, 0)),
            pl.BlockSpec((1, DFF, D), lambda i, cm: (cm[i], 0, 0)),
            pl.BlockSpec((1, D, DFF), lambda i, cm: (cm[i], 0, 0)),
            pl.BlockSpec((1, 1, BLK), lambda i, cm: (i, 0, 0)),
        ],
        out_specs=pl.BlockSpec((BLK, D), lambda i, cm: (i, 0)),
    )
    return pl.pallas_call(
        _mlp_body,
        grid_spec=grid_spec,
        out_shape=jax.ShapeDtypeStruct((NP, D), jnp.float32),
    )(cmap, disp, w1, w3, w2, wd3)


@functools.partial(
    pl.kernel,
    mesh=_mesh,
    out_type=jax.ShapeDtypeStruct((T, D), jnp.float32),
    scratch_types=[
        pltpu.VMEM((CCH,), jnp.int32),
        pltpu.VMEM((CCH, D), jnp.float32),
        pltpu.SemaphoreType.DMA,
    ],
)
def _combine_kernel(ys_hbm, rank_hbm, out_hbm, idxv, rowsv, sem):
    wid = lax.axis_index("s") * NC + lax.axis_index("c")
    for p in range(CPW // CCH):
        base = wid * CPW + p * CCH
        pltpu.sync_copy(rank_hbm.at[pl.ds(base, CCH)], idxv)
        pltpu.async_copy(ys_hbm.at[idxv], rowsv, sem).wait()
        pltpu.sync_copy(rowsv, out_hbm.at[pl.ds(base, CCH)])


def kernel(hidden_states, router_logits, per_expert_scale, w1, w3, w2):
    rank, sids, wdisp, cmap = _routing_kernel(router_logits, per_expert_scale)
    disp = _dispatch_kernel(hidden_states, sids)
    ys = _mlp_call(cmap, disp, w1, w3, w2,
                   wdisp.reshape(NCHUNK, 1, BLK))
    return _combine_kernel(ys, rank)


# speedup vs baseline: 4.7115x; 4.7115x over previous
import functools

import jax
import jax.numpy as jnp
from jax import lax
from jax.experimental import pallas as pl
from jax.experimental.pallas import tpu as pltpu
from jax.experimental.pallas import tpu_sc as plsc

E = 64
D = 1024
DFF = 1024
T = 4096
BLK = 128
NCHUNK = 96
NP = NCHUNK * BLK
NTILE = 16
TPT = T // NTILE
NGROUP = TPT // 16
NC = 2
NW = NC * NTILE
GPW = NP // NW
GCH = 96
CPW = T // NW
CCH = 64

_mesh = plsc.VectorSubcoreMesh(core_axis_name="c", subcore_axis_name="s")


def _i32(x):
    return jnp.asarray(x, jnp.int32)


def _lane():
    return lax.iota(jnp.int32, 16)


@functools.partial(
    pl.kernel,
    mesh=_mesh,
    compiler_params=pltpu.CompilerParams(needs_layout_passes=False),
    out_type=[
        jax.ShapeDtypeStruct((T,), jnp.int32),
        jax.ShapeDtypeStruct((NP,), jnp.int32),
        jax.ShapeDtypeStruct((NP,), jnp.float32),
        jax.ShapeDtypeStruct((NCHUNK,), jnp.int32),
    ],
    scratch_types=[
        pltpu.VMEM((TPT * E,), jnp.float32),
        pltpu.VMEM((E,), jnp.float32),
        pltpu.VMEM((TPT,), jnp.int32),
        pltpu.VMEM((TPT,), jnp.int32),
        pltpu.VMEM((TPT,), jnp.float32),
        pltpu.VMEM((TPT,), jnp.int32),
        pltpu.VMEM((E,), jnp.int32),
        pltpu.VMEM((NTILE * E,), jnp.int32),
        pltpu.VMEM((E,), jnp.int32),
        pltpu.VMEM((E,), jnp.int32),
        pltpu.VMEM((NP // NTILE,), jnp.int32),
        pltpu.VMEM((NP // NTILE,), jnp.float32),
        pltpu.VMEM((128,), jnp.int32),
        pltpu.VMEM((128,), jnp.int32),
        pltpu.VMEM((128,), jnp.float32),
        pltpu.VMEM((NCHUNK,), jnp.int32),
        pltpu.VMEM_SHARED((NTILE * E,), jnp.int32),
        pltpu.VMEM_SHARED((NP,), jnp.int32),
        pltpu.VMEM_SHARED((NP,), jnp.float32),
    ],
)
def _routing_kernel(logits_hbm, scale_hbm, rank_hbm, sids_hbm, wdisp_hbm,
                    cmap_hbm, lbuf, scalebuf, top1buf, rlocbuf, wtbuf,
                    rankbuf, cntbuf, allcnt, mybase, endbuf, zbuf_i, zbuf_f,
                    idxbuf, idsbuf, wsrcbuf, cmapbuf, sh_cnt, sh_sids,
                    sh_wdisp):
    c = lax.axis_index("c")
    s = lax.axis_index("s")

    @pl.when(c == 0)
    def _():
        lane = _lane()
        zpt = NP // NTILE

        pltpu.sync_copy(logits_hbm.at[pl.ds(s * TPT * E, TPT * E)], lbuf)
        pltpu.sync_copy(scale_hbm, scalebuf)

        for k in range(E // 16):
            cntbuf[pl.ds(k * 16, 16)] = jnp.zeros((16,), jnp.int32)
        for k in range(zpt // 16):
            zbuf_i[pl.ds(k * 16, 16)] = jnp.zeros((16,), jnp.int32)
            zbuf_f[pl.ds(k * 16, 16)] = jnp.zeros((16,), jnp.float32)
        pltpu.sync_copy(zbuf_i, sh_sids.at[pl.ds(s * zpt, zpt)])
        pltpu.sync_copy(zbuf_f, sh_wdisp.at[pl.ds(s * zpt, zpt)])

        def group_body(g, _):
            rows = g * 16 + lane
            m0 = jnp.full((16,), -jnp.inf, jnp.float32)
            be0 = jnp.zeros((16,), jnp.int32)

            def exp_body(e, carry):
                m, be = carry
                v = plsc.load_gather(lbuf, [rows * E + e])
                upd = v > m
                return jnp.where(upd, v, m), jnp.where(upd, e, be)

            m, be = lax.fori_loop(0, E, exp_body, (m0, be0))
            top1buf[pl.ds(g * 16, 16)] = be
            wtbuf[pl.ds(g * 16, 16)] = plsc.load_gather(scalebuf, [be])

            wrk = jnp.zeros((16,), jnp.int32)
            aft = jnp.zeros((16,), jnp.int32)
            for j in range(16):
                bj = plsc.load_gather(
                    top1buf, [jnp.broadcast_to(g * 16 + j, (16,))])
                eqv = be == bj
                wrk = wrk + jnp.where(eqv & (lane > j), 1, 0)
                aft = aft + jnp.where(eqv & (lane < j), 1, 0)
            base = plsc.load_gather(cntbuf, [be])
            rloc = base + wrk
            plsc.store_scatter(cntbuf, [be], rloc + 1, mask=aft == 0)
            rlocbuf[pl.ds(g * 16, 16)] = rloc
            return 0

        lax.fori_loop(0, NGROUP, group_body, 0)

        pltpu.sync_copy(cntbuf, sh_cnt.at[pl.ds(s * E, E)])
        plsc.subcore_barrier()
        pltpu.sync_copy(sh_cnt, allcnt)

        carry = jnp.zeros((), jnp.int32)
        for k in range(E // 16):
            sl = pl.ds(k * 16, 16)
            tot = jnp.zeros((16,), jnp.int32)
            tpre = jnp.zeros((16,), jnp.int32)
            for t in range(NTILE):
                v = allcnt[pl.ds(t * E + k * 16, 16)]
                tot = tot + v
                tpre = tpre + jnp.where(_i32(t) < s, v, 0)
            padded = ((tot + (BLK - 1)) >> 7) << 7
            cum = jnp.cumsum(padded)
            excl = cum - padded + carry
            carry = carry + jnp.sum(padded)
            mybase[sl] = excl + tpre
            endbuf[sl] = excl + padded

        def rank_body(g, _):
            sl = pl.ds(g * 16, 16)
            be = top1buf[sl]
            rankbuf[sl] = plsc.load_gather(mybase, [be]) + rlocbuf[sl]
            return 0

        lax.fori_loop(0, NGROUP, rank_body, 0)
        pltpu.sync_copy(rankbuf, rank_hbm.at[pl.ds(s * TPT, TPT)])

        plsc.subcore_barrier()
        for h in range(TPT // 128):
            for q in range(8):
                src = pl.ds(h * 128 + q * 16, 16)
                dst = pl.ds(q * 16, 16)
                idxbuf[dst] = rankbuf[src]
                idsbuf[dst] = s * TPT + h * 128 + q * 16 + lane
                wsrcbuf[dst] = wtbuf[src]
            pltpu.sync_copy(idsbuf, sh_sids.at[idxbuf])
            pltpu.sync_copy(wsrcbuf, sh_wdisp.at[idxbuf])
        plsc.subcore_barrier()
        pltpu.sync_copy(sh_sids.at[pl.ds(s * zpt, zpt)], zbuf_i)
        pltpu.sync_copy(zbuf_i, sids_hbm.at[pl.ds(s * zpt, zpt)])
        pltpu.sync_copy(sh_wdisp.at[pl.ds(s * zpt, zpt)], zbuf_f)
        pltpu.sync_copy(zbuf_f, wdisp_hbm.at[pl.ds(s * zpt, zpt)])

        @pl.when(s == 0)
        def _():
            ends = []
            for k in range(E // 16):
                ev = endbuf[pl.ds(k * 16, 16)]
                for j in range(16):
                    ends.append(jnp.max(jnp.where(lane == j, ev, 0)))
            for cv in range(NCHUNK // 16):
                nxt = (cv * 16 + lane + 1) * BLK
                acc = jnp.zeros((16,), jnp.int32)
                for e_s in ends:
                    acc = acc + jnp.where(e_s < nxt, 1, 0)
                cmapbuf[pl.ds(cv * 16, 16)] = jnp.minimum(acc, E - 1)
            pltpu.sync_copy(cmapbuf, cmap_hbm)


@functools.partial(
    pl.kernel,
    mesh=_mesh,
    out_type=jax.ShapeDtypeStruct((NP, D), jnp.float32),
    scratch_types=[
        pltpu.VMEM((GCH,), jnp.int32),
        pltpu.VMEM((GCH, D), jnp.float32),
        pltpu.SemaphoreType.DMA,
    ],
)
def _dispatch_kernel(hid_hbm, sids_hbm, disp_hbm, idxv, rowsv, sem):
    wid = lax.axis_index("s") * NC + lax.axis_index("c")
    for p in range(GPW // GCH):
        base = wid * GPW + p * GCH
        pltpu.sync_copy(sids_hbm.at[pl.ds(base, GCH)], idxv)
        pltpu.async_copy(hid_hbm.at[idxv], rowsv, sem).wait()
        pltpu.sync_copy(rowsv, disp_hbm.at[pl.ds(base, GCH)])


def _mlp_body(cmap_ref, x_ref, w1_ref, w3_ref, w2_ref, wd_ref, o_ref):
    x = x_ref[...]
    a = lax.dot_general(x, w1_ref[0], (((1,), (1,)), ((), ())),
                        preferred_element_type=jnp.float32)
    b = lax.dot_general(x, w3_ref[0], (((1,), (1,)), ((), ())),
                        preferred_element_type=jnp.float32)
    h = jax.nn.gelu(a) * b
    y = lax.dot_general(h, w2_ref[0], (((1,), (1,)), ((), ())),
                        preferred_element_type=jnp.float32)
    o_ref[...] = y * wd_ref[0, 0][:, None]


def _mlp_call(cmap, disp, w1, w3, w2, wd3):
    grid_spec = pltpu.PrefetchScalarGridSpec(
        num_scalar_prefetch=1,
        grid=(NCHUNK,),
        in_specs=[
            pl.BlockSpec((BLK, D), lambda i, cm: (i, 0)),
            pl.BlockSpec((1, DFF, D), lambda i, cm: (cm[i], 0, 0)),
            pl.BlockSpec((1, DFF, D), lambda i, cm: (cm[i], 0, 0)),
            pl.BlockSpec((1, D, DFF), lambda i, cm: (cm[i], 0, 0)),
            pl.BlockSpec((1, 1, BLK), lambda i, cm: (i, 0, 0)),
        ],
        out_specs=pl.BlockSpec((BLK, D), lambda i, cm: (i, 0)),
    )
    return pl.pallas_call(
        _mlp_body,
        grid_spec=grid_spec,
        out_shape=jax.ShapeDtypeStruct((NP, D), jnp.float32),
    )(cmap, disp, w1, w3, w2, wd3)


@functools.partial(
    pl.kernel,
    mesh=_mesh,
    out_type=jax.ShapeDtypeStruct((T, D), jnp.float32),
    scratch_types=[
        pltpu.VMEM((CCH,), jnp.int32),
        pltpu.VMEM((CCH, D), jnp.float32),
        pltpu.SemaphoreType.DMA,
    ],
)
def _combine_kernel(ys_hbm, rank_hbm, out_hbm, idxv, rowsv, sem):
    wid = lax.axis_index("s") * NC + lax.axis_index("c")
    for p in range(CPW // CCH):
        base = wid * CPW + p * CCH
        pltpu.sync_copy(rank_hbm.at[pl.ds(base, CCH)], idxv)
        pltpu.async_copy(ys_hbm.at[idxv], rowsv, sem).wait()
        pltpu.sync_copy(rowsv, out_hbm.at[pl.ds(base, CCH)])


def kernel(hidden_states, router_logits, per_expert_scale, w1, w3, w2):
    rank, sids, wdisp, cmap = _routing_kernel(
        router_logits.reshape(T * E), per_expert_scale)
    disp = _dispatch_kernel(hidden_states, sids)
    ys = _mlp_call(cmap, disp, w1, w3, w2,
                   wdisp.reshape(NCHUNK, 1, BLK))
    return _combine_kernel(ys, rank)


# speedup vs baseline: 5.3283x; 1.1309x over previous
import functools

import jax
import jax.numpy as jnp
from jax import lax
from jax.experimental import pallas as pl
from jax.experimental.pallas import tpu as pltpu
from jax.experimental.pallas import tpu_sc as plsc

E = 64
D = 1024
DFF = 1024
T = 4096
BLK = 64
BLKSH = 6
NCHUNK = 128
NP = NCHUNK * BLK
NTILE = 16
TPT = T // NTILE
NGROUP = TPT // 16
NC = 2
NW = NC * NTILE
GPW = NP // NW
GCH = 32
CPW = T // NW
CCH = 64

_mesh = plsc.VectorSubcoreMesh(core_axis_name="c", subcore_axis_name="s")


def _i32(x):
    return jnp.asarray(x, jnp.int32)


def _lane():
    return lax.iota(jnp.int32, 16)


@functools.partial(
    pl.kernel,
    mesh=_mesh,
    compiler_params=pltpu.CompilerParams(needs_layout_passes=False),
    out_type=[
        jax.ShapeDtypeStruct((T,), jnp.int32),
        jax.ShapeDtypeStruct((NP,), jnp.int32),
        jax.ShapeDtypeStruct((NP,), jnp.float32),
        jax.ShapeDtypeStruct((NCHUNK,), jnp.int32),
    ],
    scratch_types=[
        pltpu.VMEM((TPT * E,), jnp.float32),
        pltpu.VMEM((E,), jnp.float32),
        pltpu.VMEM((TPT,), jnp.int32),
        pltpu.VMEM((TPT,), jnp.int32),
        pltpu.VMEM((TPT,), jnp.float32),
        pltpu.VMEM((TPT,), jnp.int32),
        pltpu.VMEM((E,), jnp.int32),
        pltpu.VMEM((NTILE * E,), jnp.int32),
        pltpu.VMEM((E,), jnp.int32),
        pltpu.VMEM((E,), jnp.int32),
        pltpu.VMEM((NP // NTILE,), jnp.int32),
        pltpu.VMEM((NP // NTILE,), jnp.float32),
        pltpu.VMEM((128,), jnp.int32),
        pltpu.VMEM((128,), jnp.int32),
        pltpu.VMEM((128,), jnp.float32),
        pltpu.VMEM((NCHUNK,), jnp.int32),
        pltpu.VMEM_SHARED((NTILE * E,), jnp.int32),
        pltpu.VMEM_SHARED((NP,), jnp.int32),
        pltpu.VMEM_SHARED((NP,), jnp.float32),
    ],
)
def _routing_kernel(logits_hbm, scale_hbm, rank_hbm, sids_hbm, wdisp_hbm,
                    cmap_hbm, lbuf, scalebuf, top1buf, rlocbuf, wtbuf,
                    rankbuf, cntbuf, allcnt, mybase, endbuf, zbuf_i, zbuf_f,
                    idxbuf, idsbuf, wsrcbuf, cmapbuf, sh_cnt, sh_sids,
                    sh_wdisp):
    c = lax.axis_index("c")
    s = lax.axis_index("s")

    @pl.when(c == 0)
    def _():
        lane = _lane()
        zpt = NP // NTILE

        pltpu.sync_copy(logits_hbm.at[pl.ds(s * TPT * E, TPT * E)], lbuf)
        pltpu.sync_copy(scale_hbm, scalebuf)

        for k in range(E // 16):
            cntbuf[pl.ds(k * 16, 16)] = jnp.zeros((16,), jnp.int32)
        for k in range(zpt // 16):
            zbuf_i[pl.ds(k * 16, 16)] = jnp.zeros((16,), jnp.int32)
            zbuf_f[pl.ds(k * 16, 16)] = jnp.zeros((16,), jnp.float32)
        pltpu.sync_copy(zbuf_i, sh_sids.at[pl.ds(s * zpt, zpt)])
        pltpu.sync_copy(zbuf_f, sh_wdisp.at[pl.ds(s * zpt, zpt)])

        def group_body(g, _):
            rows = g * 16 + lane
            m0 = jnp.full((16,), -jnp.inf, jnp.float32)
            be0 = jnp.zeros((16,), jnp.int32)

            def exp_body(e, carry):
                m, be = carry
                v = plsc.load_gather(lbuf, [rows * E + e])
                upd = v > m
                return jnp.where(upd, v, m), jnp.where(upd, e, be)

            m, be = lax.fori_loop(0, E, exp_body, (m0, be0))
            top1buf[pl.ds(g * 16, 16)] = be
            wtbuf[pl.ds(g * 16, 16)] = plsc.load_gather(scalebuf, [be])

            wrk = jnp.zeros((16,), jnp.int32)
            aft = jnp.zeros((16,), jnp.int32)
            for j in range(16):
                bj = plsc.load_gather(
                    top1buf, [jnp.broadcast_to(g * 16 + j, (16,))])
                eqv = be == bj
                wrk = wrk + jnp.where(eqv & (lane > j), 1, 0)
                aft = aft + jnp.where(eqv & (lane < j), 1, 0)
            base = plsc.load_gather(cntbuf, [be])
            rloc = base + wrk
            plsc.store_scatter(cntbuf, [be], rloc + 1, mask=aft == 0)
            rlocbuf[pl.ds(g * 16, 16)] = rloc
            return 0

        lax.fori_loop(0, NGROUP, group_body, 0)

        pltpu.sync_copy(cntbuf, sh_cnt.at[pl.ds(s * E, E)])
        plsc.subcore_barrier()
        pltpu.sync_copy(sh_cnt, allcnt)

        carry = jnp.zeros((), jnp.int32)
        for k in range(E // 16):
            sl = pl.ds(k * 16, 16)
            tot = jnp.zeros((16,), jnp.int32)
            tpre = jnp.zeros((16,), jnp.int32)
            for t in range(NTILE):
                v = allcnt[pl.ds(t * E + k * 16, 16)]
                tot = tot + v
                tpre = tpre + jnp.where(_i32(t) < s, v, 0)
            padded = ((tot + (BLK - 1)) >> BLKSH) << BLKSH
            cum = jnp.cumsum(padded)
            excl = cum - padded + carry
            carry = carry + jnp.sum(padded)
            mybase[sl] = excl + tpre
            endbuf[sl] = excl + padded

        def rank_body(g, _):
            sl = pl.ds(g * 16, 16)
            be = top1buf[sl]
            rankbuf[sl] = plsc.load_gather(mybase, [be]) + rlocbuf[sl]
            return 0

        lax.fori_loop(0, NGROUP, rank_body, 0)
        pltpu.sync_copy(rankbuf, rank_hbm.at[pl.ds(s * TPT, TPT)])

        plsc.subcore_barrier()
        for h in range(TPT // 128):
            for q in range(8):
                src = pl.ds(h * 128 + q * 16, 16)
                dst = pl.ds(q * 16, 16)
                idxbuf[dst] = rankbuf[src]
                idsbuf[dst] = s * TPT + h * 128 + q * 16 + lane
                wsrcbuf[dst] = wtbuf[src]
            pltpu.sync_copy(idsbuf, sh_sids.at[idxbuf])
            pltpu.sync_copy(wsrcbuf, sh_wdisp.at[idxbuf])
        plsc.subcore_barrier()
        pltpu.sync_copy(sh_sids.at[pl.ds(s * zpt, zpt)], zbuf_i)
        pltpu.sync_copy(zbuf_i, sids_hbm.at[pl.ds(s * zpt, zpt)])
        pltpu.sync_copy(sh_wdisp.at[pl.ds(s * zpt, zpt)], zbuf_f)
        pltpu.sync_copy(zbuf_f, wdisp_hbm.at[pl.ds(s * zpt, zpt)])

        @pl.when(s == 0)
        def _():
            ends = []
            for k in range(E // 16):
                ev = endbuf[pl.ds(k * 16, 16)]
                for j in range(16):
                    ends.append(jnp.max(jnp.where(lane == j, ev, 0)))
            for cv in range(NCHUNK // 16):
                nxt = (cv * 16 + lane + 1) * BLK
                acc = jnp.zeros((16,), jnp.int32)
                for e_s in ends:
                    acc = acc + jnp.where(e_s < nxt, 1, 0)
                cmapbuf[pl.ds(cv * 16, 16)] = jnp.minimum(acc, E - 1)
            pltpu.sync_copy(cmapbuf, cmap_hbm)


@functools.partial(
    pl.kernel,
    mesh=_mesh,
    out_type=jax.ShapeDtypeStruct((NP, D), jnp.float32),
    scratch_types=[
        pltpu.VMEM((GPW,), jnp.int32),
        pltpu.VMEM((GCH, D), jnp.float32),
        pltpu.VMEM((GCH, D), jnp.float32),
        pltpu.SemaphoreType.DMA,
        pltpu.SemaphoreType.DMA,
        pltpu.SemaphoreType.DMA,
        pltpu.SemaphoreType.DMA,
    ],
)
def _dispatch_kernel(hid_hbm, sids_hbm, disp_hbm, allidx, rows0, rows1,
                     gsem0, gsem1, wsem0, wsem1):
    wid = lax.axis_index("s") * NC + lax.axis_index("c")
    base = wid * GPW
    npiece = GPW // GCH
    rows = (rows0, rows1)
    gsems = (gsem0, gsem1)
    wsems = (wsem0, wsem1)
    pltpu.sync_copy(sids_hbm.at[pl.ds(base, GPW)], allidx)
    gathers = [None] * npiece
    writes = [None] * npiece
    gathers[0] = pltpu.async_copy(
        hid_hbm.at[allidx.at[pl.ds(0, GCH)]], rows[0], gsems[0])
    for p in range(npiece):
        b = p & 1
        gathers[p].wait()
        writes[p] = pltpu.async_copy(
            rows[b], disp_hbm.at[pl.ds(base + p * GCH, GCH)], wsems[b])
        if p + 1 < npiece:
            nb = (p + 1) & 1
            if writes[p - 1] is not None and p >= 1:
                writes[p - 1].wait()
            gathers[p + 1] = pltpu.async_copy(
                hid_hbm.at[allidx.at[pl.ds((p + 1) * GCH, GCH)]],
                rows[nb], gsems[nb])
    writes[npiece - 2].wait()
    writes[npiece - 1].wait()


def _mlp_body(cmap_ref, x_ref, w1_ref, w3_ref, w2_ref, wd_ref, o_ref):
    x = x_ref[...]
    a = lax.dot_general(x, w1_ref[0], (((1,), (1,)), ((), ())),
                        preferred_element_type=jnp.float32)
    b = lax.dot_general(x, w3_ref[0], (((1,), (1,)), ((), ())),
                        preferred_element_type=jnp.float32)
    h = jax.nn.gelu(a) * b
    y = lax.dot_general(h, w2_ref[0], (((1,), (1,)), ((), ())),
                        preferred_element_type=jnp.float32)
    o_ref[...] = y * wd_ref[0, 0][:, None]


def _mlp_call(cmap, disp, w1, w3, w2, wd3):
    grid_spec = pltpu.PrefetchScalarGridSpec(
        num_scalar_prefetch=1,
        grid=(NCHUNK,),
        in_specs=[
            pl.BlockSpec((BLK, D), lambda i, cm: (i, 0)),
            pl.BlockSpec((1, DFF, D), lambda i, cm: (cm[i], 0, 0)),
            pl.BlockSpec((1, DFF, D), lambda i, cm: (cm[i], 0, 0)),
            pl.BlockSpec((1, D, DFF), lambda i, cm: (cm[i], 0, 0)),
            pl.BlockSpec((1, 1, BLK), lambda i, cm: (i, 0, 0)),
        ],
        out_specs=pl.BlockSpec((BLK, D), lambda i, cm: (i, 0)),
    )
    return pl.pallas_call(
        _mlp_body,
        grid_spec=grid_spec,
        out_shape=jax.ShapeDtypeStruct((NP, D), jnp.float32),
    )(cmap, disp, w1, w3, w2, wd3)


@functools.partial(
    pl.kernel,
    mesh=_mesh,
    out_type=jax.ShapeDtypeStruct((T, D), jnp.float32),
    scratch_types=[
        pltpu.VMEM((CCH,), jnp.int32),
        pltpu.VMEM((CCH, D), jnp.float32),
        pltpu.SemaphoreType.DMA,
    ],
)
def _combine_kernel(ys_hbm, rank_hbm, out_hbm, idxv, rowsv, sem):
    wid = lax.axis_index("s") * NC + lax.axis_index("c")
    for p in range(CPW // CCH):
        base = wid * CPW + p * CCH
        pltpu.sync_copy(rank_hbm.at[pl.ds(base, CCH)], idxv)
        pltpu.async_copy(ys_hbm.at[idxv], rowsv, sem).wait()
        pltpu.sync_copy(rowsv, out_hbm.at[pl.ds(base, CCH)])


def kernel(hidden_states, router_logits, per_expert_scale, w1, w3, w2):
    rank, sids, wdisp, cmap = _routing_kernel(
        router_logits.reshape(T * E), per_expert_scale)
    disp = _dispatch_kernel(hidden_states, sids)
    ys = _mlp_call(cmap, disp, w1, w3, w2,
                   wdisp.reshape(NCHUNK, 1, BLK))
    return _combine_kernel(ys, rank)


# speedup vs baseline: 7.5012x; 1.4078x over previous
import functools

import jax
import jax.numpy as jnp
from jax import lax
from jax.experimental import pallas as pl
from jax.experimental.pallas import tpu as pltpu
from jax.experimental.pallas import tpu_sc as plsc

E = 64
D = 1024
DFF = 1024
T = 4096
BLK = 64
BLKSH = 6
NCHUNK = 128
NP = NCHUNK * BLK
NTILE = 16
TPT = T // NTILE
NGROUP = TPT // 16
NC = 2
NW = NC * NTILE
GPW = NP // NW
GCH = 32
CPW = T // NW
CCH = 64

_mesh = plsc.VectorSubcoreMesh(core_axis_name="c", subcore_axis_name="s")


def _i32(x):
    return jnp.asarray(x, jnp.int32)


def _lane():
    return lax.iota(jnp.int32, 16)


@functools.partial(
    pl.kernel,
    mesh=_mesh,
    compiler_params=pltpu.CompilerParams(needs_layout_passes=False),
    out_type=[
        jax.ShapeDtypeStruct((T,), jnp.int32),
        jax.ShapeDtypeStruct((NP,), jnp.float32),
        jax.ShapeDtypeStruct((NCHUNK,), jnp.int32),
    ],
    scratch_types=[
        pltpu.VMEM((TPT * E,), jnp.float32),
        pltpu.VMEM((E,), jnp.float32),
        pltpu.VMEM((TPT,), jnp.int32),
        pltpu.VMEM((TPT,), jnp.int32),
        pltpu.VMEM((TPT,), jnp.float32),
        pltpu.VMEM((TPT,), jnp.int32),
        pltpu.VMEM((E,), jnp.int32),
        pltpu.VMEM((NTILE * E,), jnp.int32),
        pltpu.VMEM((E,), jnp.int32),
        pltpu.VMEM((E,), jnp.int32),
        pltpu.VMEM((NP // NTILE,), jnp.float32),
        pltpu.VMEM((128,), jnp.int32),
        pltpu.VMEM((128,), jnp.float32),
        pltpu.VMEM((NCHUNK,), jnp.int32),
        pltpu.VMEM_SHARED((NTILE * E,), jnp.int32),
        pltpu.VMEM_SHARED((NP,), jnp.float32),
    ],
)
def _routing_kernel(logits_hbm, scale_hbm, rank_hbm, wdisp_hbm,
                    cmap_hbm, lbuf, scalebuf, top1buf, rlocbuf, wtbuf,
                    rankbuf, cntbuf, allcnt, mybase, endbuf, zbuf_f,
                    idxbuf, wsrcbuf, cmapbuf, sh_cnt, sh_wdisp):
    c = lax.axis_index("c")
    s = lax.axis_index("s")

    @pl.when(c == 0)
    def _():
        lane = _lane()
        zpt = NP // NTILE

        pltpu.sync_copy(logits_hbm.at[pl.ds(s * TPT * E, TPT * E)], lbuf)
        pltpu.sync_copy(scale_hbm, scalebuf)

        for k in range(E // 16):
            cntbuf[pl.ds(k * 16, 16)] = jnp.zeros((16,), jnp.int32)
        for k in range(zpt // 16):
            zbuf_f[pl.ds(k * 16, 16)] = jnp.zeros((16,), jnp.float32)
        pltpu.sync_copy(zbuf_f, sh_wdisp.at[pl.ds(s * zpt, zpt)])

        def group_body(g, _):
            rows = g * 16 + lane
            m0 = jnp.full((16,), -jnp.inf, jnp.float32)
            be0 = jnp.zeros((16,), jnp.int32)

            def exp_body(e, carry):
                m, be = carry
                v = plsc.load_gather(lbuf, [rows * E + e])
                upd = v > m
                return jnp.where(upd, v, m), jnp.where(upd, e, be)

            m, be = lax.fori_loop(0, E, exp_body, (m0, be0))
            top1buf[pl.ds(g * 16, 16)] = be
            wtbuf[pl.ds(g * 16, 16)] = plsc.load_gather(scalebuf, [be])

            wrk = jnp.zeros((16,), jnp.int32)
            aft = jnp.zeros((16,), jnp.int32)
            for j in range(16):
                bj = plsc.load_gather(
                    top1buf, [jnp.broadcast_to(g * 16 + j, (16,))])
                eqv = be == bj
                wrk = wrk + jnp.where(eqv & (lane > j), 1, 0)
                aft = aft + jnp.where(eqv & (lane < j), 1, 0)
            base = plsc.load_gather(cntbuf, [be])
            rloc = base + wrk
            plsc.store_scatter(cntbuf, [be], rloc + 1, mask=aft == 0)
            rlocbuf[pl.ds(g * 16, 16)] = rloc
            return 0

        lax.fori_loop(0, NGROUP, group_body, 0)

        pltpu.sync_copy(cntbuf, sh_cnt.at[pl.ds(s * E, E)])
        plsc.subcore_barrier()
        pltpu.sync_copy(sh_cnt, allcnt)

        carry = jnp.zeros((), jnp.int32)
        for k in range(E // 16):
            sl = pl.ds(k * 16, 16)
            tot = jnp.zeros((16,), jnp.int32)
            tpre = jnp.zeros((16,), jnp.int32)
            for t in range(NTILE):
                v = allcnt[pl.ds(t * E + k * 16, 16)]
                tot = tot + v
                tpre = tpre + jnp.where(_i32(t) < s, v, 0)
            padded = ((tot + (BLK - 1)) >> BLKSH) << BLKSH
            cum = jnp.cumsum(padded)
            excl = cum - padded + carry
            carry = carry + jnp.sum(padded)
            mybase[sl] = excl + tpre
            endbuf[sl] = excl + padded

        def rank_body(g, _):
            sl = pl.ds(g * 16, 16)
            be = top1buf[sl]
            rankbuf[sl] = plsc.load_gather(mybase, [be]) + rlocbuf[sl]
            return 0

        lax.fori_loop(0, NGROUP, rank_body, 0)
        pltpu.sync_copy(rankbuf, rank_hbm.at[pl.ds(s * TPT, TPT)])

        plsc.subcore_barrier()
        for h in range(TPT // 128):
            for q in range(8):
                src = pl.ds(h * 128 + q * 16, 16)
                dst = pl.ds(q * 16, 16)
                idxbuf[dst] = rankbuf[src]
                wsrcbuf[dst] = wtbuf[src]
            pltpu.sync_copy(wsrcbuf, sh_wdisp.at[idxbuf])
        plsc.subcore_barrier()
        pltpu.sync_copy(sh_wdisp.at[pl.ds(s * zpt, zpt)], zbuf_f)
        pltpu.sync_copy(zbuf_f, wdisp_hbm.at[pl.ds(s * zpt, zpt)])

        @pl.when(s == 0)
        def _():
            ends = []
            for k in range(E // 16):
                ev = endbuf[pl.ds(k * 16, 16)]
                for j in range(16):
                    ends.append(jnp.max(jnp.where(lane == j, ev, 0)))
            for cv in range(NCHUNK // 16):
                nxt = (cv * 16 + lane + 1) * BLK
                acc = jnp.zeros((16,), jnp.int32)
                for e_s in ends:
                    acc = acc + jnp.where(e_s < nxt, 1, 0)
                cmapbuf[pl.ds(cv * 16, 16)] = jnp.minimum(acc, E - 1)
            pltpu.sync_copy(cmapbuf, cmap_hbm)


@functools.partial(
    pl.kernel,
    mesh=_mesh,
    out_type=jax.ShapeDtypeStruct((NP, D), jnp.float32),
    scratch_types=[
        pltpu.VMEM((CPW,), jnp.int32),
        pltpu.VMEM((GCH,), jnp.int32),
        pltpu.VMEM((GCH,), jnp.int32),
        pltpu.VMEM((GCH, D), jnp.float32),
        pltpu.VMEM((GCH, D), jnp.float32),
        pltpu.SemaphoreType.DMA,
        pltpu.SemaphoreType.DMA,
        pltpu.SemaphoreType.DMA,
        pltpu.SemaphoreType.DMA,
    ],
)
def _dispatch_kernel(hid_hbm, rank_hbm, disp_hbm, allidx, idx0, idx1,
                     rows0, rows1, lsem0, lsem1, ssem0, ssem1):
    wid = lax.axis_index("s") * NC + lax.axis_index("c")
    base = wid * CPW
    npiece = CPW // GCH
    rows = (rows0, rows1)
    idxs = (idx0, idx1)
    lsems = (lsem0, lsem1)
    ssems = (ssem0, ssem1)
    pltpu.sync_copy(rank_hbm.at[pl.ds(base, CPW)], allidx)
    loads = [None] * npiece
    scats = [None] * npiece
    loads[0] = pltpu.async_copy(
        hid_hbm.at[pl.ds(base, GCH)], rows[0], lsems[0])
    for p in range(npiece):
        b = p & 1
        for q in range(GCH // 16):
            idxs[b][pl.ds(q * 16, 16)] = allidx[pl.ds(p * GCH + q * 16, 16)]
        loads[p].wait()
        scats[p] = pltpu.async_copy(rows[b], disp_hbm.at[idxs[b]], ssems[b])
        if p + 1 < npiece:
            nb = (p + 1) & 1
            if p >= 1:
                scats[p - 1].wait()
            loads[p + 1] = pltpu.async_copy(
                hid_hbm.at[pl.ds(base + (p + 1) * GCH, GCH)],
                rows[nb], lsems[nb])
    scats[npiece - 2].wait()
    scats[npiece - 1].wait()


def _mlp_body(cmap_ref, x_ref, w1_ref, w3_ref, w2_ref, wd_ref, o_ref):
    x = x_ref[...]
    a = lax.dot_general(x, w1_ref[0], (((1,), (1,)), ((), ())),
                        preferred_element_type=jnp.float32)
    b = lax.dot_general(x, w3_ref[0], (((1,), (1,)), ((), ())),
                        preferred_element_type=jnp.float32)
    h = jax.nn.gelu(a) * b
    y = lax.dot_general(h, w2_ref[0], (((1,), (1,)), ((), ())),
                        preferred_element_type=jnp.float32)
    o_ref[...] = y * wd_ref[0, 0][:, None]


def _mlp_call(cmap, disp, w1, w3, w2, wd3):
    grid_spec = pltpu.PrefetchScalarGridSpec(
        num_scalar_prefetch=1,
        grid=(NCHUNK,),
        in_specs=[
            pl.BlockSpec((BLK, D), lambda i, cm: (i, 0)),
            pl.BlockSpec((1, DFF, D), lambda i, cm: (cm[i], 0, 0)),
            pl.BlockSpec((1, DFF, D), lambda i, cm: (cm[i], 0, 0)),
            pl.BlockSpec((1, D, DFF), lambda i, cm: (cm[i], 0, 0)),
            pl.BlockSpec((1, 1, BLK), lambda i, cm: (i, 0, 0)),
        ],
        out_specs=pl.BlockSpec((BLK, D), lambda i, cm: (i, 0)),
    )
    return pl.pallas_call(
        _mlp_body,
        grid_spec=grid_spec,
        out_shape=jax.ShapeDtypeStruct((NP, D), jnp.float32),
    )(cmap, disp, w1, w3, w2, wd3)


@functools.partial(
    pl.kernel,
    mesh=_mesh,
    out_type=jax.ShapeDtypeStruct((T, D), jnp.float32),
    scratch_types=[
        pltpu.VMEM((CCH,), jnp.int32),
        pltpu.VMEM((CCH, D), jnp.float32),
        pltpu.SemaphoreType.DMA,
    ],
)
def _combine_kernel(ys_hbm, rank_hbm, out_hbm, idxv, rowsv, sem):
    wid = lax.axis_index("s") * NC + lax.axis_index("c")
    for p in range(CPW // CCH):
        base = wid * CPW + p * CCH
        pltpu.sync_copy(rank_hbm.at[pl.ds(base, CCH)], idxv)
        pltpu.async_copy(ys_hbm.at[idxv], rowsv, sem).wait()
        pltpu.sync_copy(rowsv, out_hbm.at[pl.ds(base, CCH)])


def kernel(hidden_states, router_logits, per_expert_scale, w1, w3, w2):
    rank, wdisp, cmap = _routing_kernel(
        router_logits.reshape(T * E), per_expert_scale)
    disp = _dispatch_kernel(hidden_states, rank)
    ys = _mlp_call(cmap, disp, w1, w3, w2,
                   wdisp.reshape(NCHUNK, 1, BLK))
    return _combine_kernel(ys, rank)


# speedup vs baseline: 9.0668x; 1.2087x over previous
import functools

import jax
import jax.numpy as jnp
from jax import lax
from jax.experimental import pallas as pl
from jax.experimental.pallas import tpu as pltpu
from jax.experimental.pallas import tpu_sc as plsc

E = 64
D = 1024
DFF = 1024
T = 4096
BLK = 128
BLKSH = 7
NCHUNK = 96
NP = NCHUNK * BLK
NTILE = 16
TPT = T // NTILE
NGROUP = TPT // 16
NC = 2
NW = NC * NTILE
GPW = NP // NW
GCH = 32
CPW = T // NW
CCH = 64

_mesh = plsc.VectorSubcoreMesh(core_axis_name="c", subcore_axis_name="s")


def _i32(x):
    return jnp.asarray(x, jnp.int32)


def _lane():
    return lax.iota(jnp.int32, 16)


@functools.partial(
    pl.kernel,
    mesh=_mesh,
    compiler_params=pltpu.CompilerParams(needs_layout_passes=False),
    out_type=[
        jax.ShapeDtypeStruct((T,), jnp.int32),
        jax.ShapeDtypeStruct((NP,), jnp.float32),
        jax.ShapeDtypeStruct((NCHUNK,), jnp.int32),
    ],
    scratch_types=[
        pltpu.VMEM((TPT * E,), jnp.float32),
        pltpu.VMEM((E,), jnp.float32),
        pltpu.VMEM((TPT,), jnp.int32),
        pltpu.VMEM((TPT,), jnp.int32),
        pltpu.VMEM((TPT,), jnp.float32),
        pltpu.VMEM((TPT,), jnp.int32),
        pltpu.VMEM((E,), jnp.int32),
        pltpu.VMEM((NTILE * E,), jnp.int32),
        pltpu.VMEM((E,), jnp.int32),
        pltpu.VMEM((E,), jnp.int32),
        pltpu.VMEM((NP // NTILE,), jnp.float32),
        pltpu.VMEM((128,), jnp.int32),
        pltpu.VMEM((128,), jnp.float32),
        pltpu.VMEM((NCHUNK,), jnp.int32),
        pltpu.VMEM_SHARED((NTILE * E,), jnp.int32),
        pltpu.VMEM_SHARED((NP,), jnp.float32),
    ],
)
def _routing_kernel(logits_hbm, scale_hbm, rank_hbm, wdisp_hbm,
                    cmap_hbm, lbuf, scalebuf, top1buf, rlocbuf, wtbuf,
                    rankbuf, cntbuf, allcnt, mybase, endbuf, zbuf_f,
                    idxbuf, wsrcbuf, cmapbuf, sh_cnt, sh_wdisp):
    c = lax.axis_index("c")
    s = lax.axis_index("s")

    @pl.when(c == 0)
    def _():
        lane = _lane()
        zpt = NP // NTILE

        pltpu.sync_copy(logits_hbm.at[pl.ds(s * TPT * E, TPT * E)], lbuf)
        pltpu.sync_copy(scale_hbm, scalebuf)

        for k in range(E // 16):
            cntbuf[pl.ds(k * 16, 16)] = jnp.zeros((16,), jnp.int32)
        for k in range(zpt // 16):
            zbuf_f[pl.ds(k * 16, 16)] = jnp.zeros((16,), jnp.float32)
        pltpu.sync_copy(zbuf_f, sh_wdisp.at[pl.ds(s * zpt, zpt)])

        def group_body(g, _):
            rows = g * 16 + lane
            m0 = jnp.full((16,), -jnp.inf, jnp.float32)
            be0 = jnp.zeros((16,), jnp.int32)

            def exp_body(e, carry):
                m, be = carry
                v = plsc.load_gather(lbuf, [rows * E + e])
                upd = v > m
                return jnp.where(upd, v, m), jnp.where(upd, e, be)

            m, be = lax.fori_loop(0, E, exp_body, (m0, be0))
            top1buf[pl.ds(g * 16, 16)] = be
            wtbuf[pl.ds(g * 16, 16)] = plsc.load_gather(scalebuf, [be])

            wrk = jnp.zeros((16,), jnp.int32)
            aft = jnp.zeros((16,), jnp.int32)
            for j in range(16):
                bj = plsc.load_gather(
                    top1buf, [jnp.broadcast_to(g * 16 + j, (16,))])
                eqv = be == bj
                wrk = wrk + jnp.where(eqv & (lane > j), 1, 0)
                aft = aft + jnp.where(eqv & (lane < j), 1, 0)
            base = plsc.load_gather(cntbuf, [be])
            rloc = base + wrk
            plsc.store_scatter(cntbuf, [be], rloc + 1, mask=aft == 0)
            rlocbuf[pl.ds(g * 16, 16)] = rloc
            return 0

        lax.fori_loop(0, NGROUP, group_body, 0)

        pltpu.sync_copy(cntbuf, sh_cnt.at[pl.ds(s * E, E)])
        plsc.subcore_barrier()
        pltpu.sync_copy(sh_cnt, allcnt)

        carry = jnp.zeros((), jnp.int32)
        for k in range(E // 16):
            sl = pl.ds(k * 16, 16)
            tot = jnp.zeros((16,), jnp.int32)
            tpre = jnp.zeros((16,), jnp.int32)
            for t in range(NTILE):
                v = allcnt[pl.ds(t * E + k * 16, 16)]
                tot = tot + v
                tpre = tpre + jnp.where(_i32(t) < s, v, 0)
            padded = ((tot + (BLK - 1)) >> BLKSH) << BLKSH
            cum = jnp.cumsum(padded)
            excl = cum - padded + carry
            carry = carry + jnp.sum(padded)
            mybase[sl] = excl + tpre
            endbuf[sl] = excl + padded

        def rank_body(g, _):
            sl = pl.ds(g * 16, 16)
            be = top1buf[sl]
            rankbuf[sl] = plsc.load_gather(mybase, [be]) + rlocbuf[sl]
            return 0

        lax.fori_loop(0, NGROUP, rank_body, 0)
        pltpu.sync_copy(rankbuf, rank_hbm.at[pl.ds(s * TPT, TPT)])

        plsc.subcore_barrier()
        for h in range(TPT // 128):
            for q in range(8):
                src = pl.ds(h * 128 + q * 16, 16)
                dst = pl.ds(q * 16, 16)
                idxbuf[dst] = rankbuf[src]
                wsrcbuf[dst] = wtbuf[src]
            pltpu.sync_copy(wsrcbuf, sh_wdisp.at[idxbuf])
        plsc.subcore_barrier()
        pltpu.sync_copy(sh_wdisp.at[pl.ds(s * zpt, zpt)], zbuf_f)
        pltpu.sync_copy(zbuf_f, wdisp_hbm.at[pl.ds(s * zpt, zpt)])

        @pl.when(s == 0)
        def _():
            ends = []
            for k in range(E // 16):
                ev = endbuf[pl.ds(k * 16, 16)]
                for j in range(16):
                    ends.append(jnp.max(jnp.where(lane == j, ev, 0)))
            for cv in range(NCHUNK // 16):
                nxt = (cv * 16 + lane + 1) * BLK
                acc = jnp.zeros((16,), jnp.int32)
                for e_s in ends:
                    acc = acc + jnp.where(e_s < nxt, 1, 0)
                cmapbuf[pl.ds(cv * 16, 16)] = jnp.minimum(acc, E - 1)
            pltpu.sync_copy(cmapbuf, cmap_hbm)


@functools.partial(
    pl.kernel,
    mesh=_mesh,
    out_type=jax.ShapeDtypeStruct((NP, D), jnp.float32),
    scratch_types=[
        pltpu.VMEM((CPW,), jnp.int32),
        pltpu.VMEM((GCH,), jnp.int32),
        pltpu.VMEM((GCH,), jnp.int32),
        pltpu.VMEM((GCH, D), jnp.float32),
        pltpu.VMEM((GCH, D), jnp.float32),
        pltpu.SemaphoreType.DMA,
        pltpu.SemaphoreType.DMA,
        pltpu.SemaphoreType.DMA,
        pltpu.SemaphoreType.DMA,
    ],
)
def _dispatch_kernel(hid_hbm, rank_hbm, disp_hbm, allidx, idx0, idx1,
                     rows0, rows1, lsem0, lsem1, ssem0, ssem1):
    wid = lax.axis_index("s") * NC + lax.axis_index("c")
    base = wid * CPW
    npiece = CPW // GCH
    rows = (rows0, rows1)
    idxs = (idx0, idx1)
    lsems = (lsem0, lsem1)
    ssems = (ssem0, ssem1)
    pltpu.sync_copy(rank_hbm.at[pl.ds(base, CPW)], allidx)
    loads = [None] * npiece
    scats = [None] * npiece
    loads[0] = pltpu.async_copy(
        hid_hbm.at[pl.ds(base, GCH)], rows[0], lsems[0])
    for p in range(npiece):
        b = p & 1
        for q in range(GCH // 16):
            idxs[b][pl.ds(q * 16, 16)] = allidx[pl.ds(p * GCH + q * 16, 16)]
        loads[p].wait()
        scats[p] = pltpu.async_copy(rows[b], disp_hbm.at[idxs[b]], ssems[b])
        if p + 1 < npiece:
            nb = (p + 1) & 1
            if p >= 1:
                scats[p - 1].wait()
            loads[p + 1] = pltpu.async_copy(
                hid_hbm.at[pl.ds(base + (p + 1) * GCH, GCH)],
                rows[nb], lsems[nb])
    scats[npiece - 2].wait()
    scats[npiece - 1].wait()


def _mlp_body(cmap_ref, x_ref, w1_ref, w3_ref, w2_ref, wd_ref, o_ref):
    x = x_ref[...]
    a = lax.dot_general(x, w1_ref[0], (((1,), (1,)), ((), ())),
                        preferred_element_type=jnp.float32)
    b = lax.dot_general(x, w3_ref[0], (((1,), (1,)), ((), ())),
                        preferred_element_type=jnp.float32)
    h = jax.nn.gelu(a) * b
    y = lax.dot_general(h, w2_ref[0], (((1,), (1,)), ((), ())),
                        preferred_element_type=jnp.float32)
    o_ref[...] = y * wd_ref[0, 0][:, None]


def _mlp_call(cmap, disp, w1, w3, w2, wd3):
    grid_spec = pltpu.PrefetchScalarGridSpec(
        num_scalar_prefetch=1,
        grid=(NCHUNK,),
        in_specs=[
            pl.BlockSpec((BLK, D), lambda i, cm: (i, 0)),
            pl.BlockSpec((1, DFF, D), lambda i, cm: (cm[i], 0, 0)),
            pl.BlockSpec((1, DFF, D), lambda i, cm: (cm[i], 0, 0)),
            pl.BlockSpec((1, D, DFF), lambda i, cm: (cm[i], 0, 0)),
            pl.BlockSpec((1, 1, BLK), lambda i, cm: (i, 0, 0)),
        ],
        out_specs=pl.BlockSpec((BLK, D), lambda i, cm: (i, 0)),
    )
    return pl.pallas_call(
        _mlp_body,
        grid_spec=grid_spec,
        out_shape=jax.ShapeDtypeStruct((NP, D), jnp.float32),
    )(cmap, disp, w1, w3, w2, wd3)


@functools.partial(
    pl.kernel,
    mesh=_mesh,
    out_type=jax.ShapeDtypeStruct((T, D), jnp.float32),
    scratch_types=[
        pltpu.VMEM((CCH,), jnp.int32),
        pltpu.VMEM((CCH, D), jnp.float32),
        pltpu.SemaphoreType.DMA,
    ],
)
def _combine_kernel(ys_hbm, rank_hbm, out_hbm, idxv, rowsv, sem):
    wid = lax.axis_index("s") * NC + lax.axis_index("c")
    for p in range(CPW // CCH):
        base = wid * CPW + p * CCH
        pltpu.sync_copy(rank_hbm.at[pl.ds(base, CCH)], idxv)
        pltpu.async_copy(ys_hbm.at[idxv], rowsv, sem).wait()
        pltpu.sync_copy(rowsv, out_hbm.at[pl.ds(base, CCH)])


def kernel(hidden_states, router_logits, per_expert_scale, w1, w3, w2):
    rank, wdisp, cmap = _routing_kernel(
        router_logits.reshape(T * E), per_expert_scale)
    disp = _dispatch_kernel(hidden_states, rank)
    ys = _mlp_call(cmap, disp, w1, w3, w2,
                   wdisp.reshape(NCHUNK, 1, BLK))
    return _combine_kernel(ys, rank)


# speedup vs baseline: 9.0700x; 1.0004x over previous
import functools

import jax
import jax.numpy as jnp
from jax import lax
from jax.experimental import pallas as pl
from jax.experimental.pallas import tpu as pltpu
from jax.experimental.pallas import tpu_sc as plsc

E = 64
D = 1024
DFF = 1024
T = 4096
BLK = 128
BLKSH = 7
NCHUNK = 96
NP = NCHUNK * BLK
NTILE = 16
TPT = T // NTILE
NGROUP = TPT // 16
NC = 2
NW = NC * NTILE
GCH = 32
CPW = T // NW
CCH = 64

_mesh = plsc.VectorSubcoreMesh(core_axis_name="c", subcore_axis_name="s")


def _i32(x):
    return jnp.asarray(x, jnp.int32)


def _lane():
    return lax.iota(jnp.int32, 16)


@functools.partial(
    pl.kernel,
    mesh=_mesh,
    compiler_params=pltpu.CompilerParams(needs_layout_passes=False),
    out_type=[
        jax.ShapeDtypeStruct((T,), jnp.int32),
        jax.ShapeDtypeStruct((NP,), jnp.float32),
        jax.ShapeDtypeStruct((NCHUNK,), jnp.int32),
    ],
    scratch_types=[
        pltpu.VMEM((TPT * E,), jnp.float32),
        pltpu.VMEM((E,), jnp.float32),
        pltpu.VMEM((TPT,), jnp.int32),
        pltpu.VMEM((TPT,), jnp.int32),
        pltpu.VMEM((TPT,), jnp.float32),
        pltpu.VMEM((TPT,), jnp.int32),
        pltpu.VMEM((E,), jnp.int32),
        pltpu.VMEM((NTILE * E,), jnp.int32),
        pltpu.VMEM((E,), jnp.int32),
        pltpu.VMEM((E,), jnp.int32),
        pltpu.VMEM((NP // NTILE,), jnp.float32),
        pltpu.VMEM((128,), jnp.int32),
        pltpu.VMEM((128,), jnp.float32),
        pltpu.VMEM((NCHUNK,), jnp.int32),
        pltpu.VMEM_SHARED((NTILE * E,), jnp.int32),
        pltpu.VMEM_SHARED((NP,), jnp.float32),
    ],
)
def _routing_kernel(logits_hbm, scale_hbm, rank_hbm, wdisp_hbm,
                    cmap_hbm, lbuf, scalebuf, top1buf, rlocbuf, wtbuf,
                    rankbuf, cntbuf, allcnt, mybase, endbuf, zbuf_f,
                    idxbuf, wsrcbuf, cmapbuf, sh_cnt, sh_wdisp):
    c = lax.axis_index("c")
    s = lax.axis_index("s")

    @pl.when(c == 0)
    def _():
        lane = _lane()
        zpt = NP // NTILE

        pltpu.sync_copy(logits_hbm.at[pl.ds(s * TPT * E, TPT * E)], lbuf)
        pltpu.sync_copy(scale_hbm, scalebuf)

        for k in range(E // 16):
            cntbuf[pl.ds(k * 16, 16)] = jnp.zeros((16,), jnp.int32)
        for k in range(zpt // 16):
            zbuf_f[pl.ds(k * 16, 16)] = jnp.zeros((16,), jnp.float32)
        pltpu.sync_copy(zbuf_f, sh_wdisp.at[pl.ds(s * zpt, zpt)])

        def group_body(g, _):
            rows = g * 16 + lane
            m0 = jnp.full((16,), -jnp.inf, jnp.float32)
            be0 = jnp.zeros((16,), jnp.int32)

            def exp_body(e, carry):
                m, be = carry
                v = plsc.load_gather(lbuf, [rows * E + e])
                upd = v > m
                return jnp.where(upd, v, m), jnp.where(upd, e, be)

            m, be = lax.fori_loop(0, E, exp_body, (m0, be0))
            top1buf[pl.ds(g * 16, 16)] = be
            wtbuf[pl.ds(g * 16, 16)] = plsc.load_gather(scalebuf, [be])

            wrk = jnp.zeros((16,), jnp.int32)
            aft = jnp.zeros((16,), jnp.int32)
            for j in range(16):
                bj = plsc.load_gather(
                    top1buf, [jnp.broadcast_to(g * 16 + j, (16,))])
                eqv = be == bj
                wrk = wrk + jnp.where(eqv & (lane > j), 1, 0)
                aft = aft + jnp.where(eqv & (lane < j), 1, 0)
            base = plsc.load_gather(cntbuf, [be])
            rloc = base + wrk
            plsc.store_scatter(cntbuf, [be], rloc + 1, mask=aft == 0)
            rlocbuf[pl.ds(g * 16, 16)] = rloc
            return 0

        lax.fori_loop(0, NGROUP, group_body, 0)

        pltpu.sync_copy(cntbuf, sh_cnt.at[pl.ds(s * E, E)])
        plsc.subcore_barrier()
        pltpu.sync_copy(sh_cnt, allcnt)

        carry = jnp.zeros((), jnp.int32)
        for k in range(E // 16):
            sl = pl.ds(k * 16, 16)
            tot = jnp.zeros((16,), jnp.int32)
            tpre = jnp.zeros((16,), jnp.int32)
            for t in range(NTILE):
                v = allcnt[pl.ds(t * E + k * 16, 16)]
                tot = tot + v
                tpre = tpre + jnp.where(_i32(t) < s, v, 0)
            padded = ((tot + (BLK - 1)) >> BLKSH) << BLKSH
            cum = jnp.cumsum(padded)
            excl = cum - padded + carry
            carry = carry + jnp.sum(padded)
            mybase[sl] = excl + tpre
            endbuf[sl] = excl + padded

        def rank_body(g, _):
            sl = pl.ds(g * 16, 16)
            be = top1buf[sl]
            rankbuf[sl] = plsc.load_gather(mybase, [be]) + rlocbuf[sl]
            return 0

        lax.fori_loop(0, NGROUP, rank_body, 0)
        pltpu.sync_copy(rankbuf, rank_hbm.at[pl.ds(s * TPT, TPT)])

        plsc.subcore_barrier()
        for h in range(TPT // 128):
            for q in range(8):
                src = pl.ds(h * 128 + q * 16, 16)
                dst = pl.ds(q * 16, 16)
                idxbuf[dst] = rankbuf[src]
                wsrcbuf[dst] = wtbuf[src]
            pltpu.sync_copy(wsrcbuf, sh_wdisp.at[idxbuf])
        plsc.subcore_barrier()
        pltpu.sync_copy(sh_wdisp.at[pl.ds(s * zpt, zpt)], zbuf_f)
        pltpu.sync_copy(zbuf_f, wdisp_hbm.at[pl.ds(s * zpt, zpt)])

        @pl.when(s == 0)
        def _():
            ends = []
            for k in range(E // 16):
                ev = endbuf[pl.ds(k * 16, 16)]
                for j in range(16):
                    ends.append(jnp.max(jnp.where(lane == j, ev, 0)))
            for cv in range(NCHUNK // 16):
                nxt = (cv * 16 + lane + 1) * BLK
                acc = jnp.zeros((16,), jnp.int32)
                for e_s in ends:
                    acc = acc + jnp.where(e_s < nxt, 1, 0)
                cmapbuf[pl.ds(cv * 16, 16)] = jnp.minimum(acc, E - 1)
            pltpu.sync_copy(cmapbuf, cmap_hbm)


@functools.partial(
    pl.kernel,
    mesh=_mesh,
    out_type=jax.ShapeDtypeStruct((NP, D), jnp.float32),
    scratch_types=[
        pltpu.VMEM((CPW,), jnp.int32),
        pltpu.VMEM((GCH,), jnp.int32),
        pltpu.VMEM((GCH,), jnp.int32),
        pltpu.VMEM((GCH, D), jnp.float32),
        pltpu.VMEM((GCH, D), jnp.float32),
        pltpu.SemaphoreType.DMA,
        pltpu.SemaphoreType.DMA,
        pltpu.SemaphoreType.DMA,
        pltpu.SemaphoreType.DMA,
    ],
)
def _dispatch_kernel(hid_hbm, rank_hbm, disp_hbm, allidx, idx0, idx1,
                     rows0, rows1, lsem0, lsem1, ssem0, ssem1):
    wid = lax.axis_index("s") * NC + lax.axis_index("c")
    base = wid * CPW
    npiece = CPW // GCH
    rows = (rows0, rows1)
    idxs = (idx0, idx1)
    lsems = (lsem0, lsem1)
    ssems = (ssem0, ssem1)
    pltpu.sync_copy(rank_hbm.at[pl.ds(base, CPW)], allidx)
    loads = [None] * npiece
    scats = [None] * npiece
    loads[0] = pltpu.async_copy(
        hid_hbm.at[pl.ds(base, GCH)], rows[0], lsems[0])
    for p in range(npiece):
        b = p & 1
        for q in range(GCH // 16):
            idxs[b][pl.ds(q * 16, 16)] = allidx[pl.ds(p * GCH + q * 16, 16)]
        loads[p].wait()
        scats[p] = pltpu.async_copy(rows[b], disp_hbm.at[idxs[b]], ssems[b])
        if p + 1 < npiece:
            nb = (p + 1) & 1
            if p >= 1:
                scats[p - 1].wait()
            loads[p + 1] = pltpu.async_copy(
                hid_hbm.at[pl.ds(base + (p + 1) * GCH, GCH)],
                rows[nb], lsems[nb])
    scats[npiece - 2].wait()
    scats[npiece - 1].wait()


def _mlp_body(cmap_ref, x_ref, w1_ref, w3_ref, w2_ref, wd_ref, o_ref):
    x = x_ref[...]
    a = lax.dot_general(x, w1_ref[0], (((1,), (1,)), ((), ())),
                        preferred_element_type=jnp.float32)
    b = lax.dot_general(x, w3_ref[0], (((1,), (1,)), ((), ())),
                        preferred_element_type=jnp.float32)
    h = jax.nn.gelu(a) * b
    y = lax.dot_general(h, w2_ref[0], (((1,), (1,)), ((), ())),
                        preferred_element_type=jnp.float32)
    o_ref[...] = y * wd_ref[0, 0][:, None]


def _mlp_call(cmap, disp, w1, w3, w2, wd3):
    grid_spec = pltpu.PrefetchScalarGridSpec(
        num_scalar_prefetch=1,
        grid=(NCHUNK,),
        in_specs=[
            pl.BlockSpec((BLK, D), lambda i, cm: (i, 0)),
            pl.BlockSpec((1, DFF, D), lambda i, cm: (cm[i], 0, 0)),
            pl.BlockSpec((1, DFF, D), lambda i, cm: (cm[i], 0, 0)),
            pl.BlockSpec((1, D, DFF), lambda i, cm: (cm[i], 0, 0)),
            pl.BlockSpec((1, 1, BLK), lambda i, cm: (i, 0, 0)),
        ],
        out_specs=pl.BlockSpec((BLK, D), lambda i, cm: (i, 0)),
    )
    return pl.pallas_call(
        _mlp_body,
        grid_spec=grid_spec,
        out_shape=jax.ShapeDtypeStruct((NP, D), jnp.float32),
    )(cmap, disp, w1, w3, w2, wd3)


@functools.partial(
    pl.kernel,
    mesh=_mesh,
    out_type=jax.ShapeDtypeStruct((T, D), jnp.float32),
    scratch_types=[
        pltpu.VMEM((CCH,), jnp.int32),
        pltpu.VMEM((CCH, D), jnp.float32),
        pltpu.SemaphoreType.DMA,
    ],
)
def _combine_kernel(ys_hbm, rank_hbm, out_hbm, idxv, rowsv, sem):
    wid = lax.axis_index("s") * NC + lax.axis_index("c")
    for p in range(CPW // CCH):
        base = wid * CPW + p * CCH
        pltpu.sync_copy(rank_hbm.at[pl.ds(base, CCH)], idxv)
        pltpu.async_copy(ys_hbm.at[idxv], rowsv, sem).wait()
        pltpu.sync_copy(rowsv, out_hbm.at[pl.ds(base, CCH)])


def kernel(hidden_states, router_logits, per_expert_scale, w1, w3, w2):
    rank, wdisp, cmap = _routing_kernel(
        router_logits.reshape(T * E), per_expert_scale)
    disp = _dispatch_kernel(hidden_states, rank)
    ys = _mlp_call(cmap, disp, w1, w3, w2,
                   wdisp.reshape(NCHUNK, 1, BLK))
    return _combine_kernel(ys, rank)
